# baseline (speedup 1.0000x reference)
import functools

import jax
import jax.numpy as jnp
from jax import lax
from jax.experimental import pallas as pl
from jax.experimental.pallas import tpu as pltpu
from jax.experimental.pallas import tpu_sc as plsc

N = 10000
E = 320000
G = 1024
NPAD = 10240
GPAD = 1032
D = 128
NC, NS, L = 2, 16, 16
CH = 80
TCH = 128
TPT = 160
EPADG = NS * TPT * TCH
NSENT = N
NPT = 624
ZB = 208
BPT = NPAD // NS
BCH = BPT // CH
GPT = G // NS

_f32 = jnp.float32
_i32 = jnp.int32


def _fill_zero(ref, nrows, width):
    @pl.loop(0, nrows)
    def _(i):
        for j in range(width // L):
            ref[i, pl.ds(j * L, L)] = jnp.zeros((L,), _f32)


def _stats_body(dstR, batF, entF, emb, degO, bcntO, entO,
                dacc, bacc, zb1, vbuf, eidx, erows, didx, didx2,
                sem, d0, d1, d2, d3):
    c = lax.axis_index("c")
    s = lax.axis_index("s")
    ds4 = [d0, d1, d2, d3]
    ebase = c * G + s * GPT
    pltpu.sync_copy(entF.at[pl.ds(ebase, GPT)], eidx)
    pltpu.async_copy(emb.at[eidx], erows, sem).wait()
    pltpu.sync_copy(erows, entO.at[pl.ds(ebase, GPT)])
    @pl.loop(0, (NPT + 32) // L)
    def _(i):
        zb1[pl.ds(i * L, L)] = jnp.zeros((L,), _f32)
    for i in range(TCH // L):
        vbuf[pl.ds(i * L, L)] = jnp.full((L,), 1.0, _f32)
    pltpu.sync_copy(zb1.at[pl.ds(0, NPT)], dacc.at[pl.ds(s * NPT, NPT)])
    pltpu.sync_copy(zb1.at[pl.ds(0, 64)], bacc.at[pl.ds(s * 64, 64)])
    @pl.when(s == NS - 1)
    def _():
        pltpu.sync_copy(zb1.at[pl.ds(0, 32)], dacc.at[pl.ds(NS * NPT, 32)])
    @pl.when(s == 0)
    def _():
        pltpu.sync_copy(zb1.at[pl.ds(0, 8)], bacc.at[pl.ds(G, 8)])
    plsc.subcore_barrier()
    tb = (c * NS + s) * TPT
    @pl.loop(0, TPT // WIN)
    def _(w):
        pltpu.sync_copy(dstR.at[pl.ds(tb + w * WIN, WIN)], didx2)
        @pl.loop(0, WIN // 4)
        def _(o):
            for b in range(4):
                @pl.when(o > 0)
                def _():
                    pltpu.make_async_copy(vbuf, dacc.at[didx2.at[0]],
                                          ds4[b]).wait()
                pltpu.async_copy(vbuf, dacc.at[didx2.at[o * 4 + b]],
                                 ds4[b], add=True)
        for b in range(4):
            pltpu.make_async_copy(vbuf, dacc.at[didx2.at[0]], ds4[b]).wait()
    @pl.loop(0, BCH)
    def _(t):
        base = c * NPAD + s * BPT + t * CH
        pltpu.sync_copy(batF.at[pl.ds(base, CH)], didx)
        pltpu.sync_copy(vbuf.at[pl.ds(0, CH)], bacc.at[didx], add=True)
    plsc.subcore_barrier()
    pltpu.sync_copy(dacc.at[pl.ds(s * NPT, NPT)], zb1.at[pl.ds(0, NPT)])
    pltpu.sync_copy(zb1.at[pl.ds(0, NPT)], degO.at[pl.ds(c * N + s * NPT, NPT)])
    @pl.when(s == NS - 1)
    def _():
        pltpu.sync_copy(dacc.at[pl.ds(NS * NPT, 16)], zb1.at[pl.ds(NPT, 16)])
        pltpu.sync_copy(zb1.at[pl.ds(NPT, 16)],
                        degO.at[pl.ds(c * N + NS * NPT, 16)])
    pltpu.sync_copy(bacc.at[pl.ds(s * 64, 64)], vbuf.at[pl.ds(0, 64)])
    pltpu.sync_copy(vbuf.at[pl.ds(0, 64)],
                    bcntO.at[pl.ds(c * GPAD + s * 64, 64)])
    @pl.when(s == 0)
    def _():
        pltpu.sync_copy(bacc.at[pl.ds(G, 8)], vbuf.at[pl.ds(64, 8)])
        pltpu.sync_copy(vbuf.at[pl.ds(64, 8)],
                        bcntO.at[pl.ds(c * GPAD + G, 8)])


@functools.cache
def _stats_call():
    mesh = plsc.VectorSubcoreMesh(core_axis_name="c", subcore_axis_name="s",
                                  num_cores=NC, num_subcores=NS)
    return pl.kernel(
        _stats_body,
        out_type=[jax.ShapeDtypeStruct((NC * N,), _f32),
                  jax.ShapeDtypeStruct((NC * GPAD,), _f32),
                  jax.ShapeDtypeStruct((NC * G, D), _f32)],
        mesh=mesh,
        scratch_types=[pltpu.VMEM_SHARED((N + 16,), _f32),
                       pltpu.VMEM_SHARED((GPAD,), _f32),
                       pltpu.VMEM((NPT + 32,), _f32),
                       pltpu.VMEM((TCH,), _f32),
                       pltpu.VMEM((GPT,), _i32),
                       pltpu.VMEM((GPT, D), _f32),
                       pltpu.VMEM((CH,), _i32),
                       pltpu.VMEM((WIN, TCH), _i32),
                       pltpu.SemaphoreType.DMA,
                       pltpu.SemaphoreType.DMA,
                       pltpu.SemaphoreType.DMA,
                       pltpu.SemaphoreType.DMA,
                       pltpu.SemaphoreType.DMA],
    )


WIN = 32


def _msg_body(htF, srcR, dstR, SO, sacc, sidx2, didx2,
              r0, r1, g0, g1, s0, s1):
    c = lax.axis_index("c")
    s = lax.axis_index("s")
    rows = [r0, r1]
    gs = [g0, g1]
    ss = [s0, s1]
    tb = (c * NS + s) * TPT
    _fill_zero(r0, TCH, D)
    for k in range(5):
        pltpu.sync_copy(r0.at[pl.ds(0, TCH)],
                        sacc.at[pl.ds(s * NPT + k * TCH, TCH)])
    @pl.when(s == NS - 1)
    def _():
        pltpu.sync_copy(r0.at[pl.ds(0, 8)], sacc.at[pl.ds(NSENT, 8)])
    plsc.subcore_barrier()
    @pl.loop(0, TPT // WIN)
    def _(w):
        pltpu.sync_copy(srcR.at[pl.ds(tb + w * WIN, WIN)], sidx2)
        pltpu.sync_copy(dstR.at[pl.ds(tb + w * WIN, WIN)], didx2)
        for b in range(2):
            pltpu.async_copy(htF.at[sidx2.at[b]], rows[b], gs[b])
        @pl.loop(0, WIN // 2 - 1)
        def _(o):
            for b in range(2):
                pltpu.make_async_copy(htF.at[sidx2.at[0]], rows[b],
                                      gs[b]).wait()
                pltpu.async_copy(rows[b], sacc.at[didx2.at[o * 2 + b]],
                                 ss[b], add=True)
            for b in range(2):
                pltpu.make_async_copy(rows[b], sacc.at[didx2.at[0]],
                                      ss[b]).wait()
                pltpu.async_copy(htF.at[sidx2.at[o * 2 + 2 + b]],
                                 rows[b], gs[b])
        for b in range(2):
            pltpu.make_async_copy(htF.at[sidx2.at[0]], rows[b], gs[b]).wait()
            pltpu.async_copy(rows[b], sacc.at[didx2.at[WIN - 2 + b]],
                             ss[b], add=True)
        for b in range(2):
            pltpu.make_async_copy(rows[b], sacc.at[didx2.at[0]], ss[b]).wait()
    plsc.subcore_barrier()
    pltpu.sync_copy(sacc.at[pl.ds(s * NPT, NPT)],
                    SO.at[pl.ds(c * N + s * NPT, NPT)])
    @pl.when(s == NS - 1)
    def _():
        pltpu.sync_copy(sacc.at[pl.ds(NS * NPT, 16)],
                        SO.at[pl.ds(c * N + NS * NPT, 16)])


@functools.cache
def _msg_call():
    mesh = plsc.VectorSubcoreMesh(core_axis_name="c", subcore_axis_name="s",
                                  num_cores=NC, num_subcores=NS)
    return pl.kernel(
        _msg_body,
        out_type=jax.ShapeDtypeStruct((NC * N, D), _f32),
        mesh=mesh,
        scratch_types=[pltpu.VMEM_SHARED((N + 8, D), _f32),
                       pltpu.VMEM((WIN, TCH), _i32),
                       pltpu.VMEM((WIN, TCH), _i32),
                       pltpu.VMEM((TCH, D), _f32),
                       pltpu.VMEM((TCH, D), _f32),
                       pltpu.SemaphoreType.DMA,
                       pltpu.SemaphoreType.DMA,
                       pltpu.SemaphoreType.DMA,
                       pltpu.SemaphoreType.DMA],
    )


def _pool_body(h2F, batF, poolO, pacc, zb, didx, rows):
    c = lax.axis_index("c")
    s = lax.axis_index("s")
    _fill_zero(zb, 64, D)
    pltpu.sync_copy(zb.at[pl.ds(0, 64)], pacc.at[pl.ds(s * 64, 64)])
    @pl.when(s == 0)
    def _():
        pltpu.sync_copy(zb.at[pl.ds(0, 8)], pacc.at[pl.ds(G, 8)])
    plsc.subcore_barrier()
    @pl.loop(0, BCH)
    def _(t):
        base = c * NPAD + s * BPT + t * CH
        pltpu.sync_copy(batF.at[pl.ds(base, CH)], didx)
        pltpu.sync_copy(h2F.at[pl.ds(base, CH)], rows)
        pltpu.sync_copy(rows, pacc.at[didx], add=True)
    plsc.subcore_barrier()
    pltpu.sync_copy(pacc.at[pl.ds(s * 64, 64)],
                    poolO.at[pl.ds(c * GPAD + s * 64, 64)])
    @pl.when(s == 0)
    def _():
        pltpu.sync_copy(pacc.at[pl.ds(G, 8)],
                        poolO.at[pl.ds(c * GPAD + G, 8)])


@functools.cache
def _pool_call():
    mesh = plsc.VectorSubcoreMesh(core_axis_name="c", subcore_axis_name="s",
                                  num_cores=NC, num_subcores=NS)
    return pl.kernel(
        _pool_body,
        out_type=jax.ShapeDtypeStruct((NC * GPAD, D), _f32),
        mesh=mesh,
        scratch_types=[pltpu.VMEM_SHARED((GPAD, D), _f32),
                       pltpu.VMEM((64, D), _f32),
                       pltpu.VMEM((CH,), _i32),
                       pltpu.VMEM((CH, D), _f32)],
    )


RB = 2000


def _prep_tc(degc, xc, atomP, W1, ht_o, dis_o):
    T = jnp.dot(atomP[...], W1[...], preferred_element_type=_f32)
    dis = lax.rsqrt(degc[...] + 1.0)
    dis_o[...] = dis
    oh = (lax.broadcasted_iota(_i32, (RB, 16), 1) == xc[...]).astype(_f32)
    ht_o[...] = dis * jnp.dot(oh, T, preferred_element_type=_f32)


def _mid_tc(S, ht, dis, b1, W2, out):
    h1 = jnp.maximum(dis[...] * (S[...] + ht[...]) + b1[...], 0.0)
    out[...] = dis[...] * jnp.dot(h1, W2[...], preferred_element_type=_f32)


def _act_tc(S, ht, dis, b2, out):
    out[...] = jnp.maximum(dis[...] * (S[...] + ht[...]) + b2[...], 0.0)


def _final_tc(P1, P2, bc1, bc2, er1, er2, eW1, eb1, eW2, eb2,
              fcW, fcb, W1g, W1e, db1, dW2, db2, dW3, db3, out):
    relu = lambda v: jnp.maximum(v, 0.0)
    dot = functools.partial(jnp.dot, preferred_element_type=_f32)
    g1 = dot(P1[...] / jnp.maximum(bc1[...], 1.0), fcW[...]) + fcb[...]
    g2 = dot(P2[...] / jnp.maximum(bc2[...], 1.0), fcW[...]) + fcb[...]
    gl = relu(g1 + g2)
    e1 = relu(er1[...])
    e1 = relu(dot(e1, eW1[...]) + eb1[...])
    e1 = relu(dot(e1, eW2[...]) + eb2[...])
    e2 = relu(er2[...])
    e2 = relu(dot(e2, eW1[...]) + eb1[...])
    e2 = relu(dot(e2, eW2[...]) + eb2[...])
    el = relu(e1 + e2)
    h = relu(dot(gl, W1g[...]) + dot(el, W1e[...]) + db1[...])
    h = relu(dot(h, dW2[...]) + db2[...])
    out[...] = dot(h, dW3[...]) + db3[...]


def _row_spec(w):
    return pl.BlockSpec((RB, w), lambda i: (i, 0))


def _full_spec(r, w):
    return pl.BlockSpec((r, w), lambda i: (0, 0))


_prep_call = pl.pallas_call(
    _prep_tc,
    grid=(NC * N // RB,),
    in_specs=[_row_spec(1), _row_spec(1), _full_spec(16, D), _full_spec(D, D)],
    out_specs=[_row_spec(D), _row_spec(1)],
    out_shape=[jax.ShapeDtypeStruct((NC * N, D), _f32),
               jax.ShapeDtypeStruct((NC * N, 1), _f32)],
)

_mid_call = pl.pallas_call(
    _mid_tc,
    grid=(NC * N // RB,),
    in_specs=[_row_spec(D), _row_spec(D), _row_spec(1),
              _full_spec(1, D), _full_spec(D, D)],
    out_specs=_row_spec(D),
    out_shape=jax.ShapeDtypeStruct((NC * N, D), _f32),
)

_act_call = pl.pallas_call(
    _act_tc,
    grid=(NC * N // RB,),
    in_specs=[_row_spec(D), _row_spec(D), _row_spec(1), _full_spec(1, D)],
    out_specs=_row_spec(D),
    out_shape=jax.ShapeDtypeStruct((NC * N, D), _f32),
)

_final_call = pl.pallas_call(
    _final_tc,
    out_shape=jax.ShapeDtypeStruct((G, D), _f32),
)


def kernel(x1, edge_index1, ent1, batch1, x2, edge_index2, ent2, batch2,
           atom_emb, gcn_W1, gcn_b1, gcn_W2, gcn_b2, fc_W, fc_b,
           ent_emb, enc_W1, enc_b1, enc_W2, enc_b2,
           dec_W1, dec_b1, dec_W2, dec_b2, dec_W3, dec_b3):
    epad_s = jnp.zeros((EPADG - E,), _i32)
    epad_d = jnp.full((EPADG - E,), NSENT, _i32)
    srcR = jnp.concatenate([edge_index1[0].astype(_i32), epad_s,
                            edge_index2[0].astype(_i32) + N, epad_s]
                           ).reshape(NC * NS * TPT, TCH)
    dstR = jnp.concatenate([edge_index1[1].astype(_i32), epad_d,
                            edge_index2[1].astype(_i32), epad_d]
                           ).reshape(NC * NS * TPT, TCH)
    pad = jnp.full((NPAD - N,), G, _i32)
    batF = jnp.concatenate([batch1.astype(_i32), pad,
                            batch2.astype(_i32), pad])
    entF = jnp.concatenate([ent1, ent2]).astype(_i32)

    degF, bcntF, entO = _stats_call()(dstR, batF, entF, ent_emb)

    degc = degF.reshape(NC * N, 1)
    xc = jnp.concatenate([x1, x2]).astype(_i32).reshape(NC * N, 1)
    atomP = jnp.pad(atom_emb, ((0, 5), (0, 0)))
    htF, disC = _prep_call(degc, xc, atomP, gcn_W1)

    S1 = _msg_call()(htF, srcR, dstR)
    ht2F = _mid_call(S1, htF, disC, gcn_b1.reshape(1, D), gcn_W2)
    S2 = _msg_call()(ht2F, srcR, dstR)
    h2F = _act_call(S2, ht2F, disC, gcn_b2.reshape(1, D))

    zpad = jnp.zeros((NPAD - N, D), _f32)
    h2P = jnp.concatenate([h2F[:N], zpad, h2F[N:], zpad])
    poolF = _pool_call()(h2P, batF)

    return _final_call(
        poolF[:G], poolF[GPAD:GPAD + G],
        bcntF[:G].reshape(G, 1), bcntF[GPAD:GPAD + G].reshape(G, 1),
        entO[:G], entO[G:],
        enc_W1, enc_b1.reshape(1, D), enc_W2, enc_b2.reshape(1, D),
        fc_W, fc_b.reshape(1, D),
        dec_W1[:D], dec_W1[D:], dec_b1.reshape(1, 2 * D),
        dec_W2, dec_b2.reshape(1, 2 * D), dec_W3, dec_b3.reshape(1, D))


# speedup vs baseline: 1.0529x; 1.0529x over previous
import functools

import jax
import jax.numpy as jnp
from jax import lax
from jax.experimental import pallas as pl
from jax.experimental.pallas import tpu as pltpu
from jax.experimental.pallas import tpu_sc as plsc

N = 10000
E = 320000
G = 1024
NPAD = 10240
GPAD = 1032
D = 128
NC, NS, L = 2, 16, 16
CH = 80
TCH = 128
TPT = 160
EPADG = NS * TPT * TCH
NSENT = N
NPT = 624
ZB = 208
BPT = NPAD // NS
BCH = BPT // CH
GPT = G // NS

_f32 = jnp.float32
_i32 = jnp.int32


def _fill_zero(ref, nrows, width):
    @pl.loop(0, nrows)
    def _(i):
        for j in range(width // L):
            ref[i, pl.ds(j * L, L)] = jnp.zeros((L,), _f32)


def _stats_body(dstR, batF, entF, emb, degO, bcntO, entO,
                dacc, bacc, zb1, vbuf, eidx, erows, didx, didx2,
                sem, d0, d1, d2, d3):
    c = lax.axis_index("c")
    s = lax.axis_index("s")
    ds4 = [d0, d1, d2, d3]
    ebase = c * G + s * GPT
    pltpu.sync_copy(entF.at[pl.ds(ebase, GPT)], eidx)
    pltpu.async_copy(emb.at[eidx], erows, sem).wait()
    pltpu.sync_copy(erows, entO.at[pl.ds(ebase, GPT)])
    @pl.loop(0, (NPT + 32) // L)
    def _(i):
        zb1[pl.ds(i * L, L)] = jnp.zeros((L,), _f32)
    for i in range(TCH // L):
        vbuf[pl.ds(i * L, L)] = jnp.full((L,), 1.0, _f32)
    pltpu.sync_copy(zb1.at[pl.ds(0, NPT)], dacc.at[pl.ds(s * NPT, NPT)])
    pltpu.sync_copy(zb1.at[pl.ds(0, 64)], bacc.at[pl.ds(s * 64, 64)])
    @pl.when(s == NS - 1)
    def _():
        pltpu.sync_copy(zb1.at[pl.ds(0, 32)], dacc.at[pl.ds(NS * NPT, 32)])
    @pl.when(s == 0)
    def _():
        pltpu.sync_copy(zb1.at[pl.ds(0, 8)], bacc.at[pl.ds(G, 8)])
    plsc.subcore_barrier()
    tb = (c * NS + s) * TPT
    @pl.loop(0, TPT // WIN)
    def _(w):
        pltpu.sync_copy(dstR.at[pl.ds(tb + w * WIN, WIN)], didx2)
        @pl.loop(0, WIN // 4)
        def _(o):
            for b in range(4):
                @pl.when(o > 0)
                def _():
                    pltpu.make_async_copy(vbuf, dacc.at[didx2.at[0]],
                                          ds4[b]).wait()
                pltpu.async_copy(vbuf, dacc.at[didx2.at[o * 4 + b]],
                                 ds4[b], add=True)
        for b in range(4):
            pltpu.make_async_copy(vbuf, dacc.at[didx2.at[0]], ds4[b]).wait()
    @pl.loop(0, BCH)
    def _(t):
        base = c * NPAD + s * BPT + t * CH
        pltpu.sync_copy(batF.at[pl.ds(base, CH)], didx)
        pltpu.sync_copy(vbuf.at[pl.ds(0, CH)], bacc.at[didx], add=True)
    plsc.subcore_barrier()
    pltpu.sync_copy(dacc.at[pl.ds(s * NPT, NPT)], zb1.at[pl.ds(0, NPT)])
    pltpu.sync_copy(zb1.at[pl.ds(0, NPT)], degO.at[pl.ds(c * N + s * NPT, NPT)])
    @pl.when(s == NS - 1)
    def _():
        pltpu.sync_copy(dacc.at[pl.ds(NS * NPT, 16)], zb1.at[pl.ds(NPT, 16)])
        pltpu.sync_copy(zb1.at[pl.ds(NPT, 16)],
                        degO.at[pl.ds(c * N + NS * NPT, 16)])
    pltpu.sync_copy(bacc.at[pl.ds(s * 64, 64)], vbuf.at[pl.ds(0, 64)])
    pltpu.sync_copy(vbuf.at[pl.ds(0, 64)],
                    bcntO.at[pl.ds(c * GPAD + s * 64, 64)])
    @pl.when(s == 0)
    def _():
        pltpu.sync_copy(bacc.at[pl.ds(G, 8)], vbuf.at[pl.ds(64, 8)])
        pltpu.sync_copy(vbuf.at[pl.ds(64, 8)],
                        bcntO.at[pl.ds(c * GPAD + G, 8)])


@functools.cache
def _stats_call():
    mesh = plsc.VectorSubcoreMesh(core_axis_name="c", subcore_axis_name="s",
                                  num_cores=NC, num_subcores=NS)
    return pl.kernel(
        _stats_body,
        out_type=[jax.ShapeDtypeStruct((NC * N,), _f32),
                  jax.ShapeDtypeStruct((NC * GPAD,), _f32),
                  jax.ShapeDtypeStruct((NC * G, D), _f32)],
        mesh=mesh,
        scratch_types=[pltpu.VMEM_SHARED((N + 16,), _f32),
                       pltpu.VMEM_SHARED((GPAD,), _f32),
                       pltpu.VMEM((NPT + 32,), _f32),
                       pltpu.VMEM((TCH,), _f32),
                       pltpu.VMEM((GPT,), _i32),
                       pltpu.VMEM((GPT, D), _f32),
                       pltpu.VMEM((CH,), _i32),
                       pltpu.VMEM((WIN, TCH), _i32),
                       pltpu.SemaphoreType.DMA,
                       pltpu.SemaphoreType.DMA,
                       pltpu.SemaphoreType.DMA,
                       pltpu.SemaphoreType.DMA,
                       pltpu.SemaphoreType.DMA],
    )


WIN = 32


def _msg_body(htF, srcR, dstR, SO, sacc, sidx2, didx2,
              r0, r1, g0, g1, s0, s1):
    c = lax.axis_index("c")
    s = lax.axis_index("s")
    rows = [r0, r1]
    gs = [g0, g1]
    ss = [s0, s1]
    tb = (c * NS + s) * TPT
    _fill_zero(r0, TCH, D)
    for k in range(5):
        pltpu.sync_copy(r0.at[pl.ds(0, TCH)],
                        sacc.at[pl.ds(s * NPT + k * TCH, TCH)])
    @pl.when(s == NS - 1)
    def _():
        pltpu.sync_copy(r0.at[pl.ds(0, 8)], sacc.at[pl.ds(NSENT, 8)])
    plsc.subcore_barrier()
    @pl.loop(0, TPT // WIN)
    def _(w):
        pltpu.sync_copy(srcR.at[pl.ds(tb + w * WIN, WIN)], sidx2)
        pltpu.sync_copy(dstR.at[pl.ds(tb + w * WIN, WIN)], didx2)
        for b in range(2):
            pltpu.async_copy(htF.at[sidx2.at[b]], rows[b], gs[b])
        @pl.loop(0, WIN // 2 - 1)
        def _(o):
            for b in range(2):
                pltpu.make_async_copy(htF.at[sidx2.at[0]], rows[b],
                                      gs[b]).wait()
                pltpu.sync_copy(rows[b], sacc.at[didx2.at[o * 2 + b]],
                                add=True)
                pltpu.async_copy(htF.at[sidx2.at[o * 2 + 2 + b]],
                                 rows[b], gs[b])
        for b in range(2):
            pltpu.make_async_copy(htF.at[sidx2.at[0]], rows[b], gs[b]).wait()
            pltpu.sync_copy(rows[b], sacc.at[didx2.at[WIN - 2 + b]],
                            add=True)
    plsc.subcore_barrier()
    pltpu.sync_copy(sacc.at[pl.ds(s * NPT, NPT)],
                    SO.at[pl.ds(c * N + s * NPT, NPT)])
    @pl.when(s == NS - 1)
    def _():
        pltpu.sync_copy(sacc.at[pl.ds(NS * NPT, 16)],
                        SO.at[pl.ds(c * N + NS * NPT, 16)])


@functools.cache
def _msg_call():
    mesh = plsc.VectorSubcoreMesh(core_axis_name="c", subcore_axis_name="s",
                                  num_cores=NC, num_subcores=NS)
    return pl.kernel(
        _msg_body,
        out_type=jax.ShapeDtypeStruct((NC * N, D), _f32),
        mesh=mesh,
        scratch_types=[pltpu.VMEM_SHARED((N + 8, D), _f32),
                       pltpu.VMEM((WIN, TCH), _i32),
                       pltpu.VMEM((WIN, TCH), _i32),
                       pltpu.VMEM((TCH, D), _f32),
                       pltpu.VMEM((TCH, D), _f32),
                       pltpu.SemaphoreType.DMA,
                       pltpu.SemaphoreType.DMA,
                       pltpu.SemaphoreType.DMA,
                       pltpu.SemaphoreType.DMA],
    )


def _pool_body(h2F, batF, poolO, pacc, zb, didx, rows):
    c = lax.axis_index("c")
    s = lax.axis_index("s")
    _fill_zero(zb, 64, D)
    pltpu.sync_copy(zb.at[pl.ds(0, 64)], pacc.at[pl.ds(s * 64, 64)])
    @pl.when(s == 0)
    def _():
        pltpu.sync_copy(zb.at[pl.ds(0, 8)], pacc.at[pl.ds(G, 8)])
    plsc.subcore_barrier()
    @pl.loop(0, BCH)
    def _(t):
        base = c * NPAD + s * BPT + t * CH
        pltpu.sync_copy(batF.at[pl.ds(base, CH)], didx)
        pltpu.sync_copy(h2F.at[pl.ds(base, CH)], rows)
        pltpu.sync_copy(rows, pacc.at[didx], add=True)
    plsc.subcore_barrier()
    pltpu.sync_copy(pacc.at[pl.ds(s * 64, 64)],
                    poolO.at[pl.ds(c * GPAD + s * 64, 64)])
    @pl.when(s == 0)
    def _():
        pltpu.sync_copy(pacc.at[pl.ds(G, 8)],
                        poolO.at[pl.ds(c * GPAD + G, 8)])


@functools.cache
def _pool_call():
    mesh = plsc.VectorSubcoreMesh(core_axis_name="c", subcore_axis_name="s",
                                  num_cores=NC, num_subcores=NS)
    return pl.kernel(
        _pool_body,
        out_type=jax.ShapeDtypeStruct((NC * GPAD, D), _f32),
        mesh=mesh,
        scratch_types=[pltpu.VMEM_SHARED((GPAD, D), _f32),
                       pltpu.VMEM((64, D), _f32),
                       pltpu.VMEM((CH,), _i32),
                       pltpu.VMEM((CH, D), _f32)],
    )


RB = 2000


def _prep_tc(degc, xc, atomP, W1, ht_o, dis_o):
    T = jnp.dot(atomP[...], W1[...], preferred_element_type=_f32)
    dis = lax.rsqrt(degc[...] + 1.0)
    dis_o[...] = dis
    oh = (lax.broadcasted_iota(_i32, (RB, 16), 1) == xc[...]).astype(_f32)
    ht_o[...] = dis * jnp.dot(oh, T, preferred_element_type=_f32)


def _mid_tc(S, ht, dis, b1, W2, out):
    h1 = jnp.maximum(dis[...] * (S[...] + ht[...]) + b1[...], 0.0)
    out[...] = dis[...] * jnp.dot(h1, W2[...], preferred_element_type=_f32)


def _act_tc(S, ht, dis, b2, out):
    out[...] = jnp.maximum(dis[...] * (S[...] + ht[...]) + b2[...], 0.0)


def _final_tc(P1, P2, bc1, bc2, er1, er2, eW1, eb1, eW2, eb2,
              fcW, fcb, W1g, W1e, db1, dW2, db2, dW3, db3, out):
    relu = lambda v: jnp.maximum(v, 0.0)
    dot = functools.partial(jnp.dot, preferred_element_type=_f32)
    g1 = dot(P1[...] / jnp.maximum(bc1[...], 1.0), fcW[...]) + fcb[...]
    g2 = dot(P2[...] / jnp.maximum(bc2[...], 1.0), fcW[...]) + fcb[...]
    gl = relu(g1 + g2)
    e1 = relu(er1[...])
    e1 = relu(dot(e1, eW1[...]) + eb1[...])
    e1 = relu(dot(e1, eW2[...]) + eb2[...])
    e2 = relu(er2[...])
    e2 = relu(dot(e2, eW1[...]) + eb1[...])
    e2 = relu(dot(e2, eW2[...]) + eb2[...])
    el = relu(e1 + e2)
    h = relu(dot(gl, W1g[...]) + dot(el, W1e[...]) + db1[...])
    h = relu(dot(h, dW2[...]) + db2[...])
    out[...] = dot(h, dW3[...]) + db3[...]


def _row_spec(w):
    return pl.BlockSpec((RB, w), lambda i: (i, 0))


def _full_spec(r, w):
    return pl.BlockSpec((r, w), lambda i: (0, 0))


_prep_call = pl.pallas_call(
    _prep_tc,
    grid=(NC * N // RB,),
    in_specs=[_row_spec(1), _row_spec(1), _full_spec(16, D), _full_spec(D, D)],
    out_specs=[_row_spec(D), _row_spec(1)],
    out_shape=[jax.ShapeDtypeStruct((NC * N, D), _f32),
               jax.ShapeDtypeStruct((NC * N, 1), _f32)],
)

_mid_call = pl.pallas_call(
    _mid_tc,
    grid=(NC * N // RB,),
    in_specs=[_row_spec(D), _row_spec(D), _row_spec(1),
              _full_spec(1, D), _full_spec(D, D)],
    out_specs=_row_spec(D),
    out_shape=jax.ShapeDtypeStruct((NC * N, D), _f32),
)

_act_call = pl.pallas_call(
    _act_tc,
    grid=(NC * N // RB,),
    in_specs=[_row_spec(D), _row_spec(D), _row_spec(1), _full_spec(1, D)],
    out_specs=_row_spec(D),
    out_shape=jax.ShapeDtypeStruct((NC * N, D), _f32),
)

_final_call = pl.pallas_call(
    _final_tc,
    out_shape=jax.ShapeDtypeStruct((G, D), _f32),
)


def kernel(x1, edge_index1, ent1, batch1, x2, edge_index2, ent2, batch2,
           atom_emb, gcn_W1, gcn_b1, gcn_W2, gcn_b2, fc_W, fc_b,
           ent_emb, enc_W1, enc_b1, enc_W2, enc_b2,
           dec_W1, dec_b1, dec_W2, dec_b2, dec_W3, dec_b3):
    epad_s = jnp.zeros((EPADG - E,), _i32)
    epad_d = jnp.full((EPADG - E,), NSENT, _i32)
    srcR = jnp.concatenate([edge_index1[0].astype(_i32), epad_s,
                            edge_index2[0].astype(_i32) + N, epad_s]
                           ).reshape(NC * NS * TPT, TCH)
    dstR = jnp.concatenate([edge_index1[1].astype(_i32), epad_d,
                            edge_index2[1].astype(_i32), epad_d]
                           ).reshape(NC * NS * TPT, TCH)
    pad = jnp.full((NPAD - N,), G, _i32)
    batF = jnp.concatenate([batch1.astype(_i32), pad,
                            batch2.astype(_i32), pad])
    entF = jnp.concatenate([ent1, ent2]).astype(_i32)

    degF, bcntF, entO = _stats_call()(dstR, batF, entF, ent_emb)

    degc = degF.reshape(NC * N, 1)
    xc = jnp.concatenate([x1, x2]).astype(_i32).reshape(NC * N, 1)
    atomP = jnp.pad(atom_emb, ((0, 5), (0, 0)))
    htF, disC = _prep_call(degc, xc, atomP, gcn_W1)

    S1 = _msg_call()(htF, srcR, dstR)
    ht2F = _mid_call(S1, htF, disC, gcn_b1.reshape(1, D), gcn_W2)
    S2 = _msg_call()(ht2F, srcR, dstR)
    h2F = _act_call(S2, ht2F, disC, gcn_b2.reshape(1, D))

    zpad = jnp.zeros((NPAD - N, D), _f32)
    h2P = jnp.concatenate([h2F[:N], zpad, h2F[N:], zpad])
    poolF = _pool_call()(h2P, batF)

    return _final_call(
        poolF[:G], poolF[GPAD:GPAD + G],
        bcntF[:G].reshape(G, 1), bcntF[GPAD:GPAD + G].reshape(G, 1),
        entO[:G], entO[G:],
        enc_W1, enc_b1.reshape(1, D), enc_W2, enc_b2.reshape(1, D),
        fc_W, fc_b.reshape(1, D),
        dec_W1[:D], dec_W1[D:], dec_b1.reshape(1, 2 * D),
        dec_W2, dec_b2.reshape(1, 2 * D), dec_W3, dec_b3.reshape(1, D))


# speedup vs baseline: 2.0539x; 1.9508x over previous
import functools

import jax
import jax.numpy as jnp
from jax import lax
from jax.experimental import pallas as pl
from jax.experimental.pallas import tpu as pltpu
from jax.experimental.pallas import tpu_sc as plsc

N = 10000
E = 320000
G = 1024
NPAD = 10240
GPAD = 1032
D = 128
NC, NS, L = 2, 16, 16
CH = 80
TCH = 128
TPT = 160
EPADG = NS * TPT * TCH
NSENT = N
NPT = 624
ZB = 208
BPT = NPAD // NS
BCH = BPT // CH
GPT = G // NS

_f32 = jnp.float32
_i32 = jnp.int32


def _fill_zero(ref, nrows, width):
    @pl.loop(0, nrows)
    def _(i):
        for j in range(width // L):
            ref[i, pl.ds(j * L, L)] = jnp.zeros((L,), _f32)


def _stats_body(dstR, batF, entF, emb, degO, bcntO, entO,
                dacc, bacc, zb1, vbuf, eidx, erows, didx, didx2,
                sem, d0, d1, d2, d3):
    c = lax.axis_index("c")
    s = lax.axis_index("s")
    ds4 = [d0, d1, d2, d3]
    ebase = c * G + s * GPT
    pltpu.sync_copy(entF.at[pl.ds(ebase, GPT)], eidx)
    pltpu.async_copy(emb.at[eidx], erows, sem).wait()
    pltpu.sync_copy(erows, entO.at[pl.ds(ebase, GPT)])
    @pl.loop(0, (NPT + 32) // L)
    def _(i):
        zb1[pl.ds(i * L, L)] = jnp.zeros((L,), _f32)
    for i in range(TCH // L):
        vbuf[pl.ds(i * L, L)] = jnp.full((L,), 1.0, _f32)
    pltpu.sync_copy(zb1.at[pl.ds(0, NPT)], dacc.at[pl.ds(s * NPT, NPT)])
    pltpu.sync_copy(zb1.at[pl.ds(0, 64)], bacc.at[pl.ds(s * 64, 64)])
    @pl.when(s == NS - 1)
    def _():
        pltpu.sync_copy(zb1.at[pl.ds(0, 32)], dacc.at[pl.ds(NS * NPT, 32)])
    @pl.when(s == 0)
    def _():
        pltpu.sync_copy(zb1.at[pl.ds(0, 8)], bacc.at[pl.ds(G, 8)])
    plsc.subcore_barrier()
    tb = (c * NS + s) * TPT
    @pl.loop(0, TPT // WIN)
    def _(w):
        pltpu.sync_copy(dstR.at[pl.ds(tb + w * WIN, WIN)], didx2)
        @pl.loop(0, WIN // 4)
        def _(o):
            for b in range(4):
                @pl.when(o > 0)
                def _():
                    pltpu.make_async_copy(vbuf, dacc.at[didx2.at[0]],
                                          ds4[b]).wait()
                pltpu.async_copy(vbuf, dacc.at[didx2.at[o * 4 + b]],
                                 ds4[b], add=True)
        for b in range(4):
            pltpu.make_async_copy(vbuf, dacc.at[didx2.at[0]], ds4[b]).wait()
    @pl.loop(0, BCH)
    def _(t):
        base = c * NPAD + s * BPT + t * CH
        pltpu.sync_copy(batF.at[pl.ds(base, CH)], didx)
        pltpu.sync_copy(vbuf.at[pl.ds(0, CH)], bacc.at[didx], add=True)
    plsc.subcore_barrier()
    pltpu.sync_copy(dacc.at[pl.ds(s * NPT, NPT)], zb1.at[pl.ds(0, NPT)])
    pltpu.sync_copy(zb1.at[pl.ds(0, NPT)], degO.at[pl.ds(c * N + s * NPT, NPT)])
    @pl.when(s == NS - 1)
    def _():
        pltpu.sync_copy(dacc.at[pl.ds(NS * NPT, 16)], zb1.at[pl.ds(NPT, 16)])
        pltpu.sync_copy(zb1.at[pl.ds(NPT, 16)],
                        degO.at[pl.ds(c * N + NS * NPT, 16)])
    pltpu.sync_copy(bacc.at[pl.ds(s * 64, 64)], vbuf.at[pl.ds(0, 64)])
    pltpu.sync_copy(vbuf.at[pl.ds(0, 64)],
                    bcntO.at[pl.ds(c * GPAD + s * 64, 64)])
    @pl.when(s == 0)
    def _():
        pltpu.sync_copy(bacc.at[pl.ds(G, 8)], vbuf.at[pl.ds(64, 8)])
        pltpu.sync_copy(vbuf.at[pl.ds(64, 8)],
                        bcntO.at[pl.ds(c * GPAD + G, 8)])


@functools.cache
def _stats_call():
    mesh = plsc.VectorSubcoreMesh(core_axis_name="c", subcore_axis_name="s",
                                  num_cores=NC, num_subcores=NS)
    return pl.kernel(
        _stats_body,
        out_type=[jax.ShapeDtypeStruct((NC * N,), _f32),
                  jax.ShapeDtypeStruct((NC * GPAD,), _f32),
                  jax.ShapeDtypeStruct((NC * G, D), _f32)],
        mesh=mesh,
        scratch_types=[pltpu.VMEM_SHARED((N + 16,), _f32),
                       pltpu.VMEM_SHARED((GPAD,), _f32),
                       pltpu.VMEM((NPT + 32,), _f32),
                       pltpu.VMEM((TCH,), _f32),
                       pltpu.VMEM((GPT,), _i32),
                       pltpu.VMEM((GPT, D), _f32),
                       pltpu.VMEM((CH,), _i32),
                       pltpu.VMEM((WIN, TCH), _i32),
                       pltpu.SemaphoreType.DMA,
                       pltpu.SemaphoreType.DMA,
                       pltpu.SemaphoreType.DMA,
                       pltpu.SemaphoreType.DMA,
                       pltpu.SemaphoreType.DMA],
    )


WIN = 32
EPT = E // NS
NCH = EPT // CH


def _msg_body(htF, srcF, dstF, SO, sacc,
              si0, si1, di0, di1, r0, r1, g0, g1):
    c = lax.axis_index("c")
    s = lax.axis_index("s")
    rows = [r0, r1]
    sidx = [si0, si1]
    didx = [di0, di1]
    gs = [g0, g1]
    _fill_zero(r0, CH, D)
    for k in range(8):
        pltpu.sync_copy(r0.at[pl.ds(0, CH)],
                        sacc.at[pl.ds(s * NPT + k * CH, CH)])
    @pl.when(s == NS - 1)
    def _():
        pltpu.sync_copy(r0.at[pl.ds(0, 8)], sacc.at[pl.ds(NSENT, 8)])
    plsc.subcore_barrier()
    eb = c * E + s * EPT
    for b in range(2):
        pltpu.sync_copy(srcF.at[pl.ds(eb + b * CH, CH)], sidx[b])
        pltpu.sync_copy(dstF.at[pl.ds(eb + b * CH, CH)], didx[b])
        pltpu.async_copy(htF.at[sidx[b]], rows[b], gs[b])
    @pl.loop(0, NCH // 2 - 1)
    def _(o):
        for b in range(2):
            pltpu.make_async_copy(htF.at[sidx[b]], rows[b], gs[b]).wait()
            pltpu.sync_copy(rows[b], sacc.at[didx[b]], add=True)
            base = eb + (o * 2 + 2 + b) * CH
            pltpu.sync_copy(srcF.at[pl.ds(base, CH)], sidx[b])
            pltpu.sync_copy(dstF.at[pl.ds(base, CH)], didx[b])
            pltpu.async_copy(htF.at[sidx[b]], rows[b], gs[b])
    for b in range(2):
        pltpu.make_async_copy(htF.at[sidx[b]], rows[b], gs[b]).wait()
        pltpu.sync_copy(rows[b], sacc.at[didx[b]], add=True)
    plsc.subcore_barrier()
    pltpu.sync_copy(sacc.at[pl.ds(s * NPT, NPT)],
                    SO.at[pl.ds(c * N + s * NPT, NPT)])
    @pl.when(s == NS - 1)
    def _():
        pltpu.sync_copy(sacc.at[pl.ds(NS * NPT, 16)],
                        SO.at[pl.ds(c * N + NS * NPT, 16)])


@functools.cache
def _msg_call():
    mesh = plsc.VectorSubcoreMesh(core_axis_name="c", subcore_axis_name="s",
                                  num_cores=NC, num_subcores=NS)
    return pl.kernel(
        _msg_body,
        out_type=jax.ShapeDtypeStruct((NC * N, D), _f32),
        mesh=mesh,
        scratch_types=[pltpu.VMEM_SHARED((N + 8, D), _f32),
                       pltpu.VMEM((CH,), _i32),
                       pltpu.VMEM((CH,), _i32),
                       pltpu.VMEM((CH,), _i32),
                       pltpu.VMEM((CH,), _i32),
                       pltpu.VMEM((CH, D), _f32),
                       pltpu.VMEM((CH, D), _f32),
                       pltpu.SemaphoreType.DMA,
                       pltpu.SemaphoreType.DMA],
    )


def _pool_body(h2F, batF, poolO, pacc, zb, didx, rows):
    c = lax.axis_index("c")
    s = lax.axis_index("s")
    _fill_zero(zb, 64, D)
    pltpu.sync_copy(zb.at[pl.ds(0, 64)], pacc.at[pl.ds(s * 64, 64)])
    @pl.when(s == 0)
    def _():
        pltpu.sync_copy(zb.at[pl.ds(0, 8)], pacc.at[pl.ds(G, 8)])
    plsc.subcore_barrier()
    @pl.loop(0, BCH)
    def _(t):
        base = c * NPAD + s * BPT + t * CH
        pltpu.sync_copy(batF.at[pl.ds(base, CH)], didx)
        pltpu.sync_copy(h2F.at[pl.ds(base, CH)], rows)
        pltpu.sync_copy(rows, pacc.at[didx], add=True)
    plsc.subcore_barrier()
    pltpu.sync_copy(pacc.at[pl.ds(s * 64, 64)],
                    poolO.at[pl.ds(c * GPAD + s * 64, 64)])
    @pl.when(s == 0)
    def _():
        pltpu.sync_copy(pacc.at[pl.ds(G, 8)],
                        poolO.at[pl.ds(c * GPAD + G, 8)])


@functools.cache
def _pool_call():
    mesh = plsc.VectorSubcoreMesh(core_axis_name="c", subcore_axis_name="s",
                                  num_cores=NC, num_subcores=NS)
    return pl.kernel(
        _pool_body,
        out_type=jax.ShapeDtypeStruct((NC * GPAD, D), _f32),
        mesh=mesh,
        scratch_types=[pltpu.VMEM_SHARED((GPAD, D), _f32),
                       pltpu.VMEM((64, D), _f32),
                       pltpu.VMEM((CH,), _i32),
                       pltpu.VMEM((CH, D), _f32)],
    )


RB = 2000


def _prep_tc(degc, xc, atomP, W1, ht_o, dis_o):
    T = jnp.dot(atomP[...], W1[...], preferred_element_type=_f32)
    dis = lax.rsqrt(degc[...] + 1.0)
    dis_o[...] = dis
    oh = (lax.broadcasted_iota(_i32, (RB, 16), 1) == xc[...]).astype(_f32)
    ht_o[...] = dis * jnp.dot(oh, T, preferred_element_type=_f32)


def _mid_tc(S, ht, dis, b1, W2, out):
    h1 = jnp.maximum(dis[...] * (S[...] + ht[...]) + b1[...], 0.0)
    out[...] = dis[...] * jnp.dot(h1, W2[...], preferred_element_type=_f32)


def _act_tc(S, ht, dis, b2, out):
    out[...] = jnp.maximum(dis[...] * (S[...] + ht[...]) + b2[...], 0.0)


def _final_tc(P1, P2, bc1, bc2, er1, er2, eW1, eb1, eW2, eb2,
              fcW, fcb, W1g, W1e, db1, dW2, db2, dW3, db3, out):
    relu = lambda v: jnp.maximum(v, 0.0)
    dot = functools.partial(jnp.dot, preferred_element_type=_f32)
    g1 = dot(P1[...] / jnp.maximum(bc1[...], 1.0), fcW[...]) + fcb[...]
    g2 = dot(P2[...] / jnp.maximum(bc2[...], 1.0), fcW[...]) + fcb[...]
    gl = relu(g1 + g2)
    e1 = relu(er1[...])
    e1 = relu(dot(e1, eW1[...]) + eb1[...])
    e1 = relu(dot(e1, eW2[...]) + eb2[...])
    e2 = relu(er2[...])
    e2 = relu(dot(e2, eW1[...]) + eb1[...])
    e2 = relu(dot(e2, eW2[...]) + eb2[...])
    el = relu(e1 + e2)
    h = relu(dot(gl, W1g[...]) + dot(el, W1e[...]) + db1[...])
    h = relu(dot(h, dW2[...]) + db2[...])
    out[...] = dot(h, dW3[...]) + db3[...]


def _row_spec(w):
    return pl.BlockSpec((RB, w), lambda i: (i, 0))


def _full_spec(r, w):
    return pl.BlockSpec((r, w), lambda i: (0, 0))


_prep_call = pl.pallas_call(
    _prep_tc,
    grid=(NC * N // RB,),
    in_specs=[_row_spec(1), _row_spec(1), _full_spec(16, D), _full_spec(D, D)],
    out_specs=[_row_spec(D), _row_spec(1)],
    out_shape=[jax.ShapeDtypeStruct((NC * N, D), _f32),
               jax.ShapeDtypeStruct((NC * N, 1), _f32)],
)

_mid_call = pl.pallas_call(
    _mid_tc,
    grid=(NC * N // RB,),
    in_specs=[_row_spec(D), _row_spec(D), _row_spec(1),
              _full_spec(1, D), _full_spec(D, D)],
    out_specs=_row_spec(D),
    out_shape=jax.ShapeDtypeStruct((NC * N, D), _f32),
)

_act_call = pl.pallas_call(
    _act_tc,
    grid=(NC * N // RB,),
    in_specs=[_row_spec(D), _row_spec(D), _row_spec(1), _full_spec(1, D)],
    out_specs=_row_spec(D),
    out_shape=jax.ShapeDtypeStruct((NC * N, D), _f32),
)

_final_call = pl.pallas_call(
    _final_tc,
    out_shape=jax.ShapeDtypeStruct((G, D), _f32),
)


def kernel(x1, edge_index1, ent1, batch1, x2, edge_index2, ent2, batch2,
           atom_emb, gcn_W1, gcn_b1, gcn_W2, gcn_b2, fc_W, fc_b,
           ent_emb, enc_W1, enc_b1, enc_W2, enc_b2,
           dec_W1, dec_b1, dec_W2, dec_b2, dec_W3, dec_b3):
    epad_d = jnp.full((EPADG - E,), NSENT, _i32)
    srcF = jnp.concatenate([edge_index1[0].astype(_i32),
                            edge_index2[0].astype(_i32) + N])
    dstF = jnp.concatenate([edge_index1[1].astype(_i32),
                            edge_index2[1].astype(_i32)])
    dstR = jnp.concatenate([dstF[:E], epad_d, dstF[E:], epad_d]
                           ).reshape(NC * NS * TPT, TCH)
    pad = jnp.full((NPAD - N,), G, _i32)
    batF = jnp.concatenate([batch1.astype(_i32), pad,
                            batch2.astype(_i32), pad])
    entF = jnp.concatenate([ent1, ent2]).astype(_i32)

    degF, bcntF, entO = _stats_call()(dstR, batF, entF, ent_emb)

    degc = degF.reshape(NC * N, 1)
    xc = jnp.concatenate([x1, x2]).astype(_i32).reshape(NC * N, 1)
    atomP = jnp.pad(atom_emb, ((0, 5), (0, 0)))
    htF, disC = _prep_call(degc, xc, atomP, gcn_W1)

    S1 = _msg_call()(htF, srcF, dstF)
    ht2F = _mid_call(S1, htF, disC, gcn_b1.reshape(1, D), gcn_W2)
    S2 = _msg_call()(ht2F, srcF, dstF)
    h2F = _act_call(S2, ht2F, disC, gcn_b2.reshape(1, D))

    zpad = jnp.zeros((NPAD - N, D), _f32)
    h2P = jnp.concatenate([h2F[:N], zpad, h2F[N:], zpad])
    poolF = _pool_call()(h2P, batF)

    return _final_call(
        poolF[:G], poolF[GPAD:GPAD + G],
        bcntF[:G].reshape(G, 1), bcntF[GPAD:GPAD + G].reshape(G, 1),
        entO[:G], entO[G:],
        enc_W1, enc_b1.reshape(1, D), enc_W2, enc_b2.reshape(1, D),
        fc_W, fc_b.reshape(1, D),
        dec_W1[:D], dec_W1[D:], dec_b1.reshape(1, 2 * D),
        dec_W2, dec_b2.reshape(1, 2 * D), dec_W3, dec_b3.reshape(1, D))


# speedup vs baseline: 2.4988x; 1.2166x over previous
import functools

import jax
import jax.numpy as jnp
from jax import lax
from jax.experimental import pallas as pl
from jax.experimental.pallas import tpu as pltpu
from jax.experimental.pallas import tpu_sc as plsc

N = 10000
E = 320000
G = 1024
NPAD = 10240
GPAD = 1032
D = 128
NC, NS, L = 2, 16, 16
CH = 80
TCH = 128
TPT = 160
EPADG = NS * TPT * TCH
NSENT = N
NPT = 624
ZB = 208
BPT = NPAD // NS
BCH = BPT // CH
GPT = G // NS

_f32 = jnp.float32
_i32 = jnp.int32


def _fill_zero(ref, nrows, width):
    @pl.loop(0, nrows)
    def _(i):
        for j in range(width // L):
            ref[i, pl.ds(j * L, L)] = jnp.zeros((L,), _f32)


def _stats_body(dstR, batF, entF, emb, degO, bcntO, entO,
                dacc, bacc, zb1, vbuf, eidx, erows, didx, didx2,
                sem, d0, d1, d2, d3):
    c = lax.axis_index("c")
    s = lax.axis_index("s")
    ds4 = [d0, d1, d2, d3]
    ebase = c * G + s * GPT
    pltpu.sync_copy(entF.at[pl.ds(ebase, GPT)], eidx)
    pltpu.async_copy(emb.at[eidx], erows, sem).wait()
    pltpu.sync_copy(erows, entO.at[pl.ds(ebase, GPT)])
    @pl.loop(0, (NPT + 32) // L)
    def _(i):
        zb1[pl.ds(i * L, L)] = jnp.zeros((L,), _f32)
    for i in range(TCH // L):
        vbuf[pl.ds(i * L, L)] = jnp.full((L,), 1.0, _f32)
    pltpu.sync_copy(zb1.at[pl.ds(0, NPT)], dacc.at[pl.ds(s * NPT, NPT)])
    pltpu.sync_copy(zb1.at[pl.ds(0, 64)], bacc.at[pl.ds(s * 64, 64)])
    @pl.when(s == NS - 1)
    def _():
        pltpu.sync_copy(zb1.at[pl.ds(0, 32)], dacc.at[pl.ds(NS * NPT, 32)])
    @pl.when(s == 0)
    def _():
        pltpu.sync_copy(zb1.at[pl.ds(0, 8)], bacc.at[pl.ds(G, 8)])
    plsc.subcore_barrier()
    tb = (c * NS + s) * TPT
    @pl.loop(0, TPT // WIN)
    def _(w):
        pltpu.sync_copy(dstR.at[pl.ds(tb + w * WIN, WIN)], didx2)
        @pl.loop(0, WIN // 4)
        def _(o):
            for b in range(4):
                @pl.when(o > 0)
                def _():
                    pltpu.make_async_copy(vbuf, dacc.at[didx2.at[0]],
                                          ds4[b]).wait()
                pltpu.async_copy(vbuf, dacc.at[didx2.at[o * 4 + b]],
                                 ds4[b], add=True)
        for b in range(4):
            pltpu.make_async_copy(vbuf, dacc.at[didx2.at[0]], ds4[b]).wait()
    @pl.loop(0, BCH)
    def _(t):
        base = c * NPAD + s * BPT + t * CH
        pltpu.sync_copy(batF.at[pl.ds(base, CH)], didx)
        pltpu.sync_copy(vbuf.at[pl.ds(0, CH)], bacc.at[didx], add=True)
    plsc.subcore_barrier()
    pltpu.sync_copy(dacc.at[pl.ds(s * NPT, NPT)], zb1.at[pl.ds(0, NPT)])
    pltpu.sync_copy(zb1.at[pl.ds(0, NPT)], degO.at[pl.ds(c * N + s * NPT, NPT)])
    @pl.when(s == NS - 1)
    def _():
        pltpu.sync_copy(dacc.at[pl.ds(NS * NPT, 16)], zb1.at[pl.ds(NPT, 16)])
        pltpu.sync_copy(zb1.at[pl.ds(NPT, 16)],
                        degO.at[pl.ds(c * N + NS * NPT, 16)])
    pltpu.sync_copy(bacc.at[pl.ds(s * 64, 64)], vbuf.at[pl.ds(0, 64)])
    pltpu.sync_copy(vbuf.at[pl.ds(0, 64)],
                    bcntO.at[pl.ds(c * GPAD + s * 64, 64)])
    @pl.when(s == 0)
    def _():
        pltpu.sync_copy(bacc.at[pl.ds(G, 8)], vbuf.at[pl.ds(64, 8)])
        pltpu.sync_copy(vbuf.at[pl.ds(64, 8)],
                        bcntO.at[pl.ds(c * GPAD + G, 8)])


@functools.cache
def _stats_call():
    mesh = plsc.VectorSubcoreMesh(core_axis_name="c", subcore_axis_name="s",
                                  num_cores=NC, num_subcores=NS)
    return pl.kernel(
        _stats_body,
        out_type=[jax.ShapeDtypeStruct((NC * N,), _f32),
                  jax.ShapeDtypeStruct((NC * GPAD,), _f32),
                  jax.ShapeDtypeStruct((NC * G, D), _f32)],
        mesh=mesh,
        scratch_types=[pltpu.VMEM_SHARED((N + 16,), _f32),
                       pltpu.VMEM_SHARED((GPAD,), _f32),
                       pltpu.VMEM((NPT + 32,), _f32),
                       pltpu.VMEM((TCH,), _f32),
                       pltpu.VMEM((GPT,), _i32),
                       pltpu.VMEM((GPT, D), _f32),
                       pltpu.VMEM((CH,), _i32),
                       pltpu.VMEM((WIN, TCH), _i32),
                       pltpu.SemaphoreType.DMA,
                       pltpu.SemaphoreType.DMA,
                       pltpu.SemaphoreType.DMA,
                       pltpu.SemaphoreType.DMA,
                       pltpu.SemaphoreType.DMA],
    )


WIN = 32
EPT = E // NS
NCH = EPT // CH
NA = 16
CN = N * NA + 16


def _cpass_body(disH, xH, srcR, dstR, CF,
                cacc, sidx2, didx2, dg2, ag2, cix2, stage,
                m0, m1, m2, m3, a0, a1, a2, a3):
    c = lax.axis_index("c")
    s = lax.axis_index("s")
    ms = [m0, m1, m2, m3]
    asem = [a0, a1, a2, a3]
    tb = (c * NS + s) * TPT
    @pl.loop(0, 63)
    def _(i):
        stage[pl.ds(i * L, L)] = jnp.zeros((L,), _f32)
    for k in range(10):
        pltpu.sync_copy(stage.at[pl.ds(0, 1000)],
                        cacc.at[pl.ds(s * 10000 + k * 1000, 1000)])
    @pl.when(s == NS - 1)
    def _():
        pltpu.sync_copy(stage.at[pl.ds(0, 16)], cacc.at[pl.ds(N * NA, 16)])
    plsc.subcore_barrier()
    @pl.loop(0, TPT // WIN)
    def _(w):
        pltpu.sync_copy(srcR.at[pl.ds(tb + w * WIN, WIN)], sidx2)
        pltpu.sync_copy(dstR.at[pl.ds(tb + w * WIN, WIN)], didx2)
        @pl.loop(0, WIN // 4)
        def _(o):
            for b in range(4):
                @pl.when(o > 0)
                def _():
                    pltpu.make_async_copy(disH.at[sidx2.at[0]],
                                          dg2.at[0], ms[b]).wait()
                    pltpu.make_async_copy(xH.at[sidx2.at[0]],
                                          ag2.at[0], asem[b]).wait()
                t = o * 4 + b
                pltpu.async_copy(disH.at[sidx2.at[t]], dg2.at[t], ms[b])
                pltpu.async_copy(xH.at[sidx2.at[t]], ag2.at[t], asem[b])
        for b in range(4):
            pltpu.make_async_copy(disH.at[sidx2.at[0]], dg2.at[0],
                                  ms[b]).wait()
            pltpu.make_async_copy(xH.at[sidx2.at[0]], ag2.at[0],
                                  asem[b]).wait()
        @pl.loop(0, WIN)
        def _(t):
            for j in range(TCH // L):
                d = didx2[t, pl.ds(j * L, L)]
                a = ag2[t, pl.ds(j * L, L)]
                cix2[t, pl.ds(j * L, L)] = d * NA + a
        @pl.loop(0, WIN // 4)
        def _(o):
            for b in range(4):
                @pl.when(o > 0)
                def _():
                    pltpu.make_async_copy(dg2.at[0], cacc.at[cix2.at[0]],
                                          ms[b]).wait()
                t = o * 4 + b
                pltpu.async_copy(dg2.at[t], cacc.at[cix2.at[t]],
                                 ms[b], add=True)
        for b in range(4):
            pltpu.make_async_copy(dg2.at[0], cacc.at[cix2.at[0]],
                                  ms[b]).wait()
    plsc.subcore_barrier()
    pltpu.sync_copy(cacc.at[pl.ds(s * 10000, 10000)], stage.at[pl.ds(0, 10000)])
    pltpu.sync_copy(stage.at[pl.ds(0, 10000)],
                    CF.at[pl.ds(c * CN + s * 10000, 10000)])
    @pl.when(s == NS - 1)
    def _():
        pltpu.sync_copy(cacc.at[pl.ds(N * NA, 16)], stage.at[pl.ds(10000, 16)])
        pltpu.sync_copy(stage.at[pl.ds(10000, 16)],
                        CF.at[pl.ds(c * CN + N * NA, 16)])


@functools.cache
def _cpass_call():
    mesh = plsc.VectorSubcoreMesh(core_axis_name="c", subcore_axis_name="s",
                                  num_cores=NC, num_subcores=NS)
    return pl.kernel(
        _cpass_body,
        out_type=jax.ShapeDtypeStruct((NC * CN,), _f32),
        mesh=mesh,
        scratch_types=[pltpu.VMEM_SHARED((CN,), _f32),
                       pltpu.VMEM((WIN, TCH), _i32),
                       pltpu.VMEM((WIN, TCH), _i32),
                       pltpu.VMEM((WIN, TCH), _f32),
                       pltpu.VMEM((WIN, TCH), _i32),
                       pltpu.VMEM((WIN, TCH), _i32),
                       pltpu.VMEM((10016,), _f32),
                       pltpu.SemaphoreType.DMA,
                       pltpu.SemaphoreType.DMA,
                       pltpu.SemaphoreType.DMA,
                       pltpu.SemaphoreType.DMA,
                       pltpu.SemaphoreType.DMA,
                       pltpu.SemaphoreType.DMA,
                       pltpu.SemaphoreType.DMA,
                       pltpu.SemaphoreType.DMA],
    )


def _msg_body(htF, srcF, dstF, SO, sacc,
              si0, si1, di0, di1, r0, r1, g0, g1):
    c = lax.axis_index("c")
    s = lax.axis_index("s")
    rows = [r0, r1]
    sidx = [si0, si1]
    didx = [di0, di1]
    gs = [g0, g1]
    _fill_zero(r0, CH, D)
    for k in range(8):
        pltpu.sync_copy(r0.at[pl.ds(0, CH)],
                        sacc.at[pl.ds(s * NPT + k * CH, CH)])
    @pl.when(s == NS - 1)
    def _():
        pltpu.sync_copy(r0.at[pl.ds(0, 8)], sacc.at[pl.ds(NSENT, 8)])
    plsc.subcore_barrier()
    eb = c * E + s * EPT
    for b in range(2):
        pltpu.sync_copy(srcF.at[pl.ds(eb + b * CH, CH)], sidx[b])
        pltpu.sync_copy(dstF.at[pl.ds(eb + b * CH, CH)], didx[b])
        pltpu.async_copy(htF.at[sidx[b]], rows[b], gs[b])
    @pl.loop(0, NCH // 2 - 1)
    def _(o):
        for b in range(2):
            pltpu.make_async_copy(htF.at[sidx[b]], rows[b], gs[b]).wait()
            pltpu.sync_copy(rows[b], sacc.at[didx[b]], add=True)
            base = eb + (o * 2 + 2 + b) * CH
            pltpu.sync_copy(srcF.at[pl.ds(base, CH)], sidx[b])
            pltpu.sync_copy(dstF.at[pl.ds(base, CH)], didx[b])
            pltpu.async_copy(htF.at[sidx[b]], rows[b], gs[b])
    for b in range(2):
        pltpu.make_async_copy(htF.at[sidx[b]], rows[b], gs[b]).wait()
        pltpu.sync_copy(rows[b], sacc.at[didx[b]], add=True)
    plsc.subcore_barrier()
    pltpu.sync_copy(sacc.at[pl.ds(s * NPT, NPT)],
                    SO.at[pl.ds(c * N + s * NPT, NPT)])
    @pl.when(s == NS - 1)
    def _():
        pltpu.sync_copy(sacc.at[pl.ds(NS * NPT, 16)],
                        SO.at[pl.ds(c * N + NS * NPT, 16)])


@functools.cache
def _msg_call():
    mesh = plsc.VectorSubcoreMesh(core_axis_name="c", subcore_axis_name="s",
                                  num_cores=NC, num_subcores=NS)
    return pl.kernel(
        _msg_body,
        out_type=jax.ShapeDtypeStruct((NC * N, D), _f32),
        mesh=mesh,
        scratch_types=[pltpu.VMEM_SHARED((N + 8, D), _f32),
                       pltpu.VMEM((CH,), _i32),
                       pltpu.VMEM((CH,), _i32),
                       pltpu.VMEM((CH,), _i32),
                       pltpu.VMEM((CH,), _i32),
                       pltpu.VMEM((CH, D), _f32),
                       pltpu.VMEM((CH, D), _f32),
                       pltpu.SemaphoreType.DMA,
                       pltpu.SemaphoreType.DMA],
    )


def _pool_body(h2F, batF, poolO, pacc, zb, didx, rows):
    c = lax.axis_index("c")
    s = lax.axis_index("s")
    _fill_zero(zb, 64, D)
    pltpu.sync_copy(zb.at[pl.ds(0, 64)], pacc.at[pl.ds(s * 64, 64)])
    @pl.when(s == 0)
    def _():
        pltpu.sync_copy(zb.at[pl.ds(0, 8)], pacc.at[pl.ds(G, 8)])
    plsc.subcore_barrier()
    @pl.loop(0, BCH)
    def _(t):
        base = c * NPAD + s * BPT + t * CH
        pltpu.sync_copy(batF.at[pl.ds(base, CH)], didx)
        pltpu.sync_copy(h2F.at[pl.ds(base, CH)], rows)
        pltpu.sync_copy(rows, pacc.at[didx], add=True)
    plsc.subcore_barrier()
    pltpu.sync_copy(pacc.at[pl.ds(s * 64, 64)],
                    poolO.at[pl.ds(c * GPAD + s * 64, 64)])
    @pl.when(s == 0)
    def _():
        pltpu.sync_copy(pacc.at[pl.ds(G, 8)],
                        poolO.at[pl.ds(c * GPAD + G, 8)])


@functools.cache
def _pool_call():
    mesh = plsc.VectorSubcoreMesh(core_axis_name="c", subcore_axis_name="s",
                                  num_cores=NC, num_subcores=NS)
    return pl.kernel(
        _pool_body,
        out_type=jax.ShapeDtypeStruct((NC * GPAD, D), _f32),
        mesh=mesh,
        scratch_types=[pltpu.VMEM_SHARED((GPAD, D), _f32),
                       pltpu.VMEM((64, D), _f32),
                       pltpu.VMEM((CH,), _i32),
                       pltpu.VMEM((CH, D), _f32)],
    )


RB = 2000


def _prep_tc(degc, xc, atomP, W1, ht_o, dis_o):
    T = jnp.dot(atomP[...], W1[...], preferred_element_type=_f32)
    dis = lax.rsqrt(degc[...] + 1.0)
    dis_o[...] = dis
    oh = (lax.broadcasted_iota(_i32, (RB, 16), 1) == xc[...]).astype(_f32)
    ht_o[...] = dis * jnp.dot(oh, T, preferred_element_type=_f32)


def _mid_tc(C, ht, dis, atomP, W1, b1, W2, out):
    T = jnp.dot(atomP[...], W1[...], preferred_element_type=_f32)
    S1 = jnp.dot(C[...], T, preferred_element_type=_f32)
    h1 = jnp.maximum(dis[...] * (S1 + ht[...]) + b1[...], 0.0)
    out[...] = dis[...] * jnp.dot(h1, W2[...], preferred_element_type=_f32)


def _act_tc(S, ht, dis, b2, out):
    out[...] = jnp.maximum(dis[...] * (S[...] + ht[...]) + b2[...], 0.0)


def _final_tc(P1, P2, bc1, bc2, er1, er2, eW1, eb1, eW2, eb2,
              fcW, fcb, W1g, W1e, db1, dW2, db2, dW3, db3, out):
    relu = lambda v: jnp.maximum(v, 0.0)
    dot = functools.partial(jnp.dot, preferred_element_type=_f32)
    g1 = dot(P1[...] / jnp.maximum(bc1[...], 1.0), fcW[...]) + fcb[...]
    g2 = dot(P2[...] / jnp.maximum(bc2[...], 1.0), fcW[...]) + fcb[...]
    gl = relu(g1 + g2)
    e1 = relu(er1[...])
    e1 = relu(dot(e1, eW1[...]) + eb1[...])
    e1 = relu(dot(e1, eW2[...]) + eb2[...])
    e2 = relu(er2[...])
    e2 = relu(dot(e2, eW1[...]) + eb1[...])
    e2 = relu(dot(e2, eW2[...]) + eb2[...])
    el = relu(e1 + e2)
    h = relu(dot(gl, W1g[...]) + dot(el, W1e[...]) + db1[...])
    h = relu(dot(h, dW2[...]) + db2[...])
    out[...] = dot(h, dW3[...]) + db3[...]


def _row_spec(w):
    return pl.BlockSpec((RB, w), lambda i: (i, 0))


def _full_spec(r, w):
    return pl.BlockSpec((r, w), lambda i: (0, 0))


_prep_call = pl.pallas_call(
    _prep_tc,
    grid=(NC * N // RB,),
    in_specs=[_row_spec(1), _row_spec(1), _full_spec(16, D), _full_spec(D, D)],
    out_specs=[_row_spec(D), _row_spec(1)],
    out_shape=[jax.ShapeDtypeStruct((NC * N, D), _f32),
               jax.ShapeDtypeStruct((NC * N, 1), _f32)],
)

_mid_call = pl.pallas_call(
    _mid_tc,
    grid=(NC * N // RB,),
    in_specs=[_row_spec(NA), _row_spec(D), _row_spec(1),
              _full_spec(16, D), _full_spec(D, D),
              _full_spec(1, D), _full_spec(D, D)],
    out_specs=_row_spec(D),
    out_shape=jax.ShapeDtypeStruct((NC * N, D), _f32),
)

_act_call = pl.pallas_call(
    _act_tc,
    grid=(NC * N // RB,),
    in_specs=[_row_spec(D), _row_spec(D), _row_spec(1), _full_spec(1, D)],
    out_specs=_row_spec(D),
    out_shape=jax.ShapeDtypeStruct((NC * N, D), _f32),
)

_final_call = pl.pallas_call(
    _final_tc,
    out_shape=jax.ShapeDtypeStruct((G, D), _f32),
)


def kernel(x1, edge_index1, ent1, batch1, x2, edge_index2, ent2, batch2,
           atom_emb, gcn_W1, gcn_b1, gcn_W2, gcn_b2, fc_W, fc_b,
           ent_emb, enc_W1, enc_b1, enc_W2, enc_b2,
           dec_W1, dec_b1, dec_W2, dec_b2, dec_W3, dec_b3):
    epad_d = jnp.full((EPADG - E,), NSENT, _i32)
    srcF = jnp.concatenate([edge_index1[0].astype(_i32),
                            edge_index2[0].astype(_i32) + N])
    dstF = jnp.concatenate([edge_index1[1].astype(_i32),
                            edge_index2[1].astype(_i32)])
    dstR = jnp.concatenate([dstF[:E], epad_d, dstF[E:], epad_d]
                           ).reshape(NC * NS * TPT, TCH)
    pad = jnp.full((NPAD - N,), G, _i32)
    batF = jnp.concatenate([batch1.astype(_i32), pad,
                            batch2.astype(_i32), pad])
    entF = jnp.concatenate([ent1, ent2]).astype(_i32)

    degF, bcntF, entO = _stats_call()(dstR, batF, entF, ent_emb)

    degc = degF.reshape(NC * N, 1)
    xc = jnp.concatenate([x1, x2]).astype(_i32).reshape(NC * N, 1)
    atomP = jnp.pad(atom_emb, ((0, 5), (0, 0)))
    htF, disC = _prep_call(degc, xc, atomP, gcn_W1)

    srcR = jnp.concatenate([srcF[:E], jnp.zeros((EPADG - E,), _i32),
                            srcF[E:], jnp.zeros((EPADG - E,), _i32)]
                           ).reshape(NC * NS * TPT, TCH)
    CF = _cpass_call()(disC.reshape(NC * N), xc.reshape(NC * N), srcR, dstR)
    Cc = jnp.concatenate([CF[:N * NA].reshape(N, NA),
                          CF[CN:CN + N * NA].reshape(N, NA)])
    ht2F = _mid_call(Cc, htF, disC, atomP, gcn_W1,
                     gcn_b1.reshape(1, D), gcn_W2)
    S2 = _msg_call()(ht2F, srcF, dstF)
    h2F = _act_call(S2, ht2F, disC, gcn_b2.reshape(1, D))

    zpad = jnp.zeros((NPAD - N, D), _f32)
    h2P = jnp.concatenate([h2F[:N], zpad, h2F[N:], zpad])
    poolF = _pool_call()(h2P, batF)

    return _final_call(
        poolF[:G], poolF[GPAD:GPAD + G],
        bcntF[:G].reshape(G, 1), bcntF[GPAD:GPAD + G].reshape(G, 1),
        entO[:G], entO[G:],
        enc_W1, enc_b1.reshape(1, D), enc_W2, enc_b2.reshape(1, D),
        fc_W, fc_b.reshape(1, D),
        dec_W1[:D], dec_W1[D:], dec_b1.reshape(1, 2 * D),
        dec_W2, dec_b2.reshape(1, 2 * D), dec_W3, dec_b3.reshape(1, D))


# speedup vs baseline: 3.1172x; 1.2475x over previous
import functools

import jax
import jax.numpy as jnp
from jax import lax
from jax.experimental import pallas as pl
from jax.experimental.pallas import tpu as pltpu
from jax.experimental.pallas import tpu_sc as plsc

N = 10000
E = 320000
G = 1024
NPAD = 10240
GPAD = 1032
D = 128
NC, NS, L = 2, 16, 16
CH = 80
TCH = 128
TPT = 160
EPADG = NS * TPT * TCH
NSENT = N
NPT = 624
ZB = 208
BPT = NPAD // NS
BCH = BPT // CH
GPT = G // NS

_f32 = jnp.float32
_i32 = jnp.int32


def _fill_zero(ref, nrows, width):
    @pl.loop(0, nrows)
    def _(i):
        for j in range(width // L):
            ref[i, pl.ds(j * L, L)] = jnp.zeros((L,), _f32)


def _stats_body(dstR, batF, entF, emb, degO, bcntO, entO,
                dacc, bacc, zb1, vbuf, eidx, erows, didx, didx2,
                sem, d0, d1, d2, d3):
    c = lax.axis_index("c")
    s = lax.axis_index("s")
    ds4 = [d0, d1, d2, d3]
    ebase = c * G + s * GPT
    pltpu.sync_copy(entF.at[pl.ds(ebase, GPT)], eidx)
    pltpu.async_copy(emb.at[eidx], erows, sem).wait()
    pltpu.sync_copy(erows, entO.at[pl.ds(ebase, GPT)])
    @pl.loop(0, (NPT + 32) // L)
    def _(i):
        zb1[pl.ds(i * L, L)] = jnp.zeros((L,), _f32)
    for i in range(TCH // L):
        vbuf[pl.ds(i * L, L)] = jnp.full((L,), 1.0, _f32)
    pltpu.sync_copy(zb1.at[pl.ds(0, NPT)], dacc.at[pl.ds(s * NPT, NPT)])
    pltpu.sync_copy(zb1.at[pl.ds(0, 64)], bacc.at[pl.ds(s * 64, 64)])
    @pl.when(s == NS - 1)
    def _():
        pltpu.sync_copy(zb1.at[pl.ds(0, 32)], dacc.at[pl.ds(NS * NPT, 32)])
    @pl.when(s == 0)
    def _():
        pltpu.sync_copy(zb1.at[pl.ds(0, 8)], bacc.at[pl.ds(G, 8)])
    plsc.subcore_barrier()
    tb = (c * NS + s) * TPT
    @pl.loop(0, TPT // WIN)
    def _(w):
        pltpu.sync_copy(dstR.at[pl.ds(tb + w * WIN, WIN)], didx2)
        @pl.loop(0, WIN // 4)
        def _(o):
            for b in range(4):
                @pl.when(o > 0)
                def _():
                    pltpu.make_async_copy(vbuf, dacc.at[didx2.at[0]],
                                          ds4[b]).wait()
                pltpu.async_copy(vbuf, dacc.at[didx2.at[o * 4 + b]],
                                 ds4[b], add=True)
        for b in range(4):
            pltpu.make_async_copy(vbuf, dacc.at[didx2.at[0]], ds4[b]).wait()
    @pl.loop(0, BCH)
    def _(t):
        base = c * NPAD + s * BPT + t * CH
        pltpu.sync_copy(batF.at[pl.ds(base, CH)], didx)
        pltpu.sync_copy(vbuf.at[pl.ds(0, CH)], bacc.at[didx], add=True)
    plsc.subcore_barrier()
    pltpu.sync_copy(dacc.at[pl.ds(s * NPT, NPT)], zb1.at[pl.ds(0, NPT)])
    pltpu.sync_copy(zb1.at[pl.ds(0, NPT)], degO.at[pl.ds(c * N + s * NPT, NPT)])
    @pl.when(s == NS - 1)
    def _():
        pltpu.sync_copy(dacc.at[pl.ds(NS * NPT, 16)], zb1.at[pl.ds(NPT, 16)])
        pltpu.sync_copy(zb1.at[pl.ds(NPT, 16)],
                        degO.at[pl.ds(c * N + NS * NPT, 16)])
    pltpu.sync_copy(bacc.at[pl.ds(s * 64, 64)], vbuf.at[pl.ds(0, 64)])
    pltpu.sync_copy(vbuf.at[pl.ds(0, 64)],
                    bcntO.at[pl.ds(c * GPAD + s * 64, 64)])
    @pl.when(s == 0)
    def _():
        pltpu.sync_copy(bacc.at[pl.ds(G, 8)], vbuf.at[pl.ds(64, 8)])
        pltpu.sync_copy(vbuf.at[pl.ds(64, 8)],
                        bcntO.at[pl.ds(c * GPAD + G, 8)])


@functools.cache
def _stats_call():
    mesh = plsc.VectorSubcoreMesh(core_axis_name="c", subcore_axis_name="s",
                                  num_cores=NC, num_subcores=NS)
    return pl.kernel(
        _stats_body,
        out_type=[jax.ShapeDtypeStruct((NC * N,), _f32),
                  jax.ShapeDtypeStruct((NC * GPAD,), _f32),
                  jax.ShapeDtypeStruct((NC * G, D), _f32)],
        mesh=mesh,
        scratch_types=[pltpu.VMEM_SHARED((N + 16,), _f32),
                       pltpu.VMEM_SHARED((GPAD,), _f32),
                       pltpu.VMEM((NPT + 32,), _f32),
                       pltpu.VMEM((TCH,), _f32),
                       pltpu.VMEM((GPT,), _i32),
                       pltpu.VMEM((GPT, D), _f32),
                       pltpu.VMEM((CH,), _i32),
                       pltpu.VMEM((WIN, TCH), _i32),
                       pltpu.SemaphoreType.DMA,
                       pltpu.SemaphoreType.DMA,
                       pltpu.SemaphoreType.DMA,
                       pltpu.SemaphoreType.DMA,
                       pltpu.SemaphoreType.DMA],
    )


WIN = 32
EPT = E // NS
NCH = EPT // CH
NA = 16
CN = N * NA + 16


def _cpass_body(disH, xH, srcR, dstR, CF,
                cacc, sidx2, didx2, dg2, ag2, cix2, stage,
                m0, m1, m2, m3, a0, a1, a2, a3):
    c = lax.axis_index("c")
    s = lax.axis_index("s")
    ms = [m0, m1, m2, m3]
    asem = [a0, a1, a2, a3]
    tb = (c * NS + s) * TPT
    @pl.loop(0, 63)
    def _(i):
        stage[pl.ds(i * L, L)] = jnp.zeros((L,), _f32)
    for k in range(10):
        pltpu.sync_copy(stage.at[pl.ds(0, 1000)],
                        cacc.at[pl.ds(s * 10000 + k * 1000, 1000)])
    @pl.when(s == NS - 1)
    def _():
        pltpu.sync_copy(stage.at[pl.ds(0, 16)], cacc.at[pl.ds(N * NA, 16)])
    plsc.subcore_barrier()
    @pl.loop(0, TPT // WIN)
    def _(w):
        pltpu.sync_copy(srcR.at[pl.ds(tb + w * WIN, WIN)], sidx2)
        pltpu.sync_copy(dstR.at[pl.ds(tb + w * WIN, WIN)], didx2)
        @pl.loop(0, WIN // 4)
        def _(o):
            for b in range(4):
                @pl.when(o > 0)
                def _():
                    pltpu.make_async_copy(disH.at[sidx2.at[0]],
                                          dg2.at[0], ms[b]).wait()
                    pltpu.make_async_copy(xH.at[sidx2.at[0]],
                                          ag2.at[0], asem[b]).wait()
                t = o * 4 + b
                pltpu.async_copy(disH.at[sidx2.at[t]], dg2.at[t], ms[b])
                pltpu.async_copy(xH.at[sidx2.at[t]], ag2.at[t], asem[b])
        for b in range(4):
            pltpu.make_async_copy(disH.at[sidx2.at[0]], dg2.at[0],
                                  ms[b]).wait()
            pltpu.make_async_copy(xH.at[sidx2.at[0]], ag2.at[0],
                                  asem[b]).wait()
        @pl.loop(0, WIN)
        def _(t):
            for j in range(TCH // L):
                d = didx2[t, pl.ds(j * L, L)]
                a = ag2[t, pl.ds(j * L, L)]
                cix2[t, pl.ds(j * L, L)] = d * NA + a
        @pl.loop(0, WIN // 4)
        def _(o):
            for b in range(4):
                @pl.when(o > 0)
                def _():
                    pltpu.make_async_copy(dg2.at[0], cacc.at[cix2.at[0]],
                                          ms[b]).wait()
                t = o * 4 + b
                pltpu.async_copy(dg2.at[t], cacc.at[cix2.at[t]],
                                 ms[b], add=True)
        for b in range(4):
            pltpu.make_async_copy(dg2.at[0], cacc.at[cix2.at[0]],
                                  ms[b]).wait()
    plsc.subcore_barrier()
    pltpu.sync_copy(cacc.at[pl.ds(s * 10000, 10000)], stage.at[pl.ds(0, 10000)])
    pltpu.sync_copy(stage.at[pl.ds(0, 10000)],
                    CF.at[pl.ds(c * CN + s * 10000, 10000)])
    @pl.when(s == NS - 1)
    def _():
        pltpu.sync_copy(cacc.at[pl.ds(N * NA, 16)], stage.at[pl.ds(10000, 16)])
        pltpu.sync_copy(stage.at[pl.ds(10000, 16)],
                        CF.at[pl.ds(c * CN + N * NA, 16)])


@functools.cache
def _cpass_call():
    mesh = plsc.VectorSubcoreMesh(core_axis_name="c", subcore_axis_name="s",
                                  num_cores=NC, num_subcores=NS)
    return pl.kernel(
        _cpass_body,
        out_type=jax.ShapeDtypeStruct((NC * CN,), _f32),
        mesh=mesh,
        scratch_types=[pltpu.VMEM_SHARED((CN,), _f32),
                       pltpu.VMEM((WIN, TCH), _i32),
                       pltpu.VMEM((WIN, TCH), _i32),
                       pltpu.VMEM((WIN, TCH), _f32),
                       pltpu.VMEM((WIN, TCH), _i32),
                       pltpu.VMEM((WIN, TCH), _i32),
                       pltpu.VMEM((10016,), _f32),
                       pltpu.SemaphoreType.DMA,
                       pltpu.SemaphoreType.DMA,
                       pltpu.SemaphoreType.DMA,
                       pltpu.SemaphoreType.DMA,
                       pltpu.SemaphoreType.DMA,
                       pltpu.SemaphoreType.DMA,
                       pltpu.SemaphoreType.DMA,
                       pltpu.SemaphoreType.DMA],
    )


def _msg_body(htF, pkF, SO, sacc,
              si0, si1, di0, di1, r0, r1, p00, p01, p10, p11,
              g0, g1, q00, q01, q10, q11):
    c = lax.axis_index("c")
    s = lax.axis_index("s")
    rows = [r0, r1]
    sidx = [si0, si1]
    didx = [di0, di1]
    gs = [g0, g1]
    pidx = [[p00, p01], [p10, p11]]
    qs = [[q00, q01], [q10, q11]]

    def unpack(b, p):
        for j in range(CH // L):
            v = pidx[b][p][pl.ds(j * L, L)]
            sidx[b][pl.ds(j * L, L)] = lax.shift_right_logical(v, 14)
            didx[b][pl.ds(j * L, L)] = lax.bitwise_and(v, 16383)
    _fill_zero(r0, CH, D)
    for k in range(8):
        pltpu.sync_copy(r0.at[pl.ds(0, CH)],
                        sacc.at[pl.ds(s * NPT + k * CH, CH)])
    @pl.when(s == NS - 1)
    def _():
        pltpu.sync_copy(r0.at[pl.ds(0, 8)], sacc.at[pl.ds(NSENT, 8)])
    plsc.subcore_barrier()
    eb = c * E + s * EPT

    def step(t_chunk, b, p, issue_idx, next_gather):
        pltpu.make_async_copy(htF.at[sidx[b]], rows[b], gs[b]).wait()
        pltpu.sync_copy(rows[b], sacc.at[didx[b]], add=True)
        if issue_idx:
            pltpu.async_copy(pkF.at[pl.ds(eb + (t_chunk + 4) * CH, CH)],
                             pidx[b][p], qs[b][p])
        if next_gather:
            pltpu.make_async_copy(pkF.at[pl.ds(eb, CH)], pidx[b][1 - p],
                                  qs[b][1 - p]).wait()
            unpack(b, 1 - p)
            pltpu.async_copy(htF.at[sidx[b]], rows[b], gs[b])

    for t in range(4):
        pltpu.async_copy(pkF.at[pl.ds(eb + t * CH, CH)],
                         pidx[t % 2][(t // 2) % 2], qs[t % 2][(t // 2) % 2])
    for b in range(2):
        pltpu.make_async_copy(pkF.at[pl.ds(eb, CH)], pidx[b][0],
                              qs[b][0]).wait()
        unpack(b, 0)
        pltpu.async_copy(htF.at[sidx[b]], rows[b], gs[b])
    @pl.loop(0, 61)
    def _(q):
        t0 = q * 4
        for k in range(4):
            step(t0 + k, k % 2, (k // 2) % 2, True, True)
    for t in range(244, 250):
        step(t, t % 2, (t // 2) % 2, t + 4 < NCH, t + 2 < NCH)
    plsc.subcore_barrier()
    pltpu.sync_copy(sacc.at[pl.ds(s * NPT, NPT)],
                    SO.at[pl.ds(c * N + s * NPT, NPT)])
    @pl.when(s == NS - 1)
    def _():
        pltpu.sync_copy(sacc.at[pl.ds(NS * NPT, 16)],
                        SO.at[pl.ds(c * N + NS * NPT, 16)])


@functools.cache
def _msg_call():
    mesh = plsc.VectorSubcoreMesh(core_axis_name="c", subcore_axis_name="s",
                                  num_cores=NC, num_subcores=NS)
    return pl.kernel(
        _msg_body,
        out_type=jax.ShapeDtypeStruct((NC * N, D), _f32),
        mesh=mesh,
        scratch_types=[pltpu.VMEM_SHARED((N + 8, D), _f32),
                       pltpu.VMEM((CH,), _i32),
                       pltpu.VMEM((CH,), _i32),
                       pltpu.VMEM((CH,), _i32),
                       pltpu.VMEM((CH,), _i32),
                       pltpu.VMEM((CH, D), _f32),
                       pltpu.VMEM((CH, D), _f32),
                       pltpu.VMEM((CH,), _i32),
                       pltpu.VMEM((CH,), _i32),
                       pltpu.VMEM((CH,), _i32),
                       pltpu.VMEM((CH,), _i32),
                       pltpu.SemaphoreType.DMA,
                       pltpu.SemaphoreType.DMA,
                       pltpu.SemaphoreType.DMA,
                       pltpu.SemaphoreType.DMA,
                       pltpu.SemaphoreType.DMA,
                       pltpu.SemaphoreType.DMA],
    )


def _pool_body(h2F, batF, poolO, pacc, zb, didx, rows):
    c = lax.axis_index("c")
    s = lax.axis_index("s")
    _fill_zero(zb, 64, D)
    pltpu.sync_copy(zb.at[pl.ds(0, 64)], pacc.at[pl.ds(s * 64, 64)])
    @pl.when(s == 0)
    def _():
        pltpu.sync_copy(zb.at[pl.ds(0, 8)], pacc.at[pl.ds(G, 8)])
    plsc.subcore_barrier()
    @pl.loop(0, BCH)
    def _(t):
        base = c * NPAD + s * BPT + t * CH
        pltpu.sync_copy(batF.at[pl.ds(base, CH)], didx)
        pltpu.sync_copy(h2F.at[pl.ds(base, CH)], rows)
        pltpu.sync_copy(rows, pacc.at[didx], add=True)
    plsc.subcore_barrier()
    pltpu.sync_copy(pacc.at[pl.ds(s * 64, 64)],
                    poolO.at[pl.ds(c * GPAD + s * 64, 64)])
    @pl.when(s == 0)
    def _():
        pltpu.sync_copy(pacc.at[pl.ds(G, 8)],
                        poolO.at[pl.ds(c * GPAD + G, 8)])


@functools.cache
def _pool_call():
    mesh = plsc.VectorSubcoreMesh(core_axis_name="c", subcore_axis_name="s",
                                  num_cores=NC, num_subcores=NS)
    return pl.kernel(
        _pool_body,
        out_type=jax.ShapeDtypeStruct((NC * GPAD, D), _f32),
        mesh=mesh,
        scratch_types=[pltpu.VMEM_SHARED((GPAD, D), _f32),
                       pltpu.VMEM((64, D), _f32),
                       pltpu.VMEM((CH,), _i32),
                       pltpu.VMEM((CH, D), _f32)],
    )


RB = 2000


def _prep_tc(degc, xc, atomP, W1, ht_o, dis_o):
    T = jnp.dot(atomP[...], W1[...], preferred_element_type=_f32)
    dis = lax.rsqrt(degc[...] + 1.0)
    dis_o[...] = dis
    oh = (lax.broadcasted_iota(_i32, (RB, 16), 1) == xc[...]).astype(_f32)
    ht_o[...] = dis * jnp.dot(oh, T, preferred_element_type=_f32)


def _mid_tc(C, ht, dis, atomP, W1, b1, W2, out):
    T = jnp.dot(atomP[...], W1[...], preferred_element_type=_f32)
    S1 = jnp.dot(C[...], T, preferred_element_type=_f32)
    h1 = jnp.maximum(dis[...] * (S1 + ht[...]) + b1[...], 0.0)
    out[...] = dis[...] * jnp.dot(h1, W2[...], preferred_element_type=_f32)


def _act_tc(S, ht, dis, b2, out):
    out[...] = jnp.maximum(dis[...] * (S[...] + ht[...]) + b2[...], 0.0)


def _final_tc(P1, P2, bc1, bc2, er1, er2, eW1, eb1, eW2, eb2,
              fcW, fcb, W1g, W1e, db1, dW2, db2, dW3, db3, out):
    relu = lambda v: jnp.maximum(v, 0.0)
    dot = functools.partial(jnp.dot, preferred_element_type=_f32)
    g1 = dot(P1[...] / jnp.maximum(bc1[...], 1.0), fcW[...]) + fcb[...]
    g2 = dot(P2[...] / jnp.maximum(bc2[...], 1.0), fcW[...]) + fcb[...]
    gl = relu(g1 + g2)
    e1 = relu(er1[...])
    e1 = relu(dot(e1, eW1[...]) + eb1[...])
    e1 = relu(dot(e1, eW2[...]) + eb2[...])
    e2 = relu(er2[...])
    e2 = relu(dot(e2, eW1[...]) + eb1[...])
    e2 = relu(dot(e2, eW2[...]) + eb2[...])
    el = relu(e1 + e2)
    h = relu(dot(gl, W1g[...]) + dot(el, W1e[...]) + db1[...])
    h = relu(dot(h, dW2[...]) + db2[...])
    out[...] = dot(h, dW3[...]) + db3[...]


def _row_spec(w):
    return pl.BlockSpec((RB, w), lambda i: (i, 0))


def _full_spec(r, w):
    return pl.BlockSpec((r, w), lambda i: (0, 0))


_prep_call = pl.pallas_call(
    _prep_tc,
    grid=(NC * N // RB,),
    in_specs=[_row_spec(1), _row_spec(1), _full_spec(16, D), _full_spec(D, D)],
    out_specs=[_row_spec(D), _row_spec(1)],
    out_shape=[jax.ShapeDtypeStruct((NC * N, D), _f32),
               jax.ShapeDtypeStruct((NC * N, 1), _f32)],
)

_mid_call = pl.pallas_call(
    _mid_tc,
    grid=(NC * N // RB,),
    in_specs=[_row_spec(NA), _row_spec(D), _row_spec(1),
              _full_spec(16, D), _full_spec(D, D),
              _full_spec(1, D), _full_spec(D, D)],
    out_specs=_row_spec(D),
    out_shape=jax.ShapeDtypeStruct((NC * N, D), _f32),
)

_act_call = pl.pallas_call(
    _act_tc,
    grid=(NC * N // RB,),
    in_specs=[_row_spec(D), _row_spec(D), _row_spec(1), _full_spec(1, D)],
    out_specs=_row_spec(D),
    out_shape=jax.ShapeDtypeStruct((NC * N, D), _f32),
)

_final_call = pl.pallas_call(
    _final_tc,
    out_shape=jax.ShapeDtypeStruct((G, D), _f32),
)


def kernel(x1, edge_index1, ent1, batch1, x2, edge_index2, ent2, batch2,
           atom_emb, gcn_W1, gcn_b1, gcn_W2, gcn_b2, fc_W, fc_b,
           ent_emb, enc_W1, enc_b1, enc_W2, enc_b2,
           dec_W1, dec_b1, dec_W2, dec_b2, dec_W3, dec_b3):
    epad_d = jnp.full((EPADG - E,), NSENT, _i32)
    srcF = jnp.concatenate([edge_index1[0].astype(_i32),
                            edge_index2[0].astype(_i32) + N])
    dstF = jnp.concatenate([edge_index1[1].astype(_i32),
                            edge_index2[1].astype(_i32)])
    dstR = jnp.concatenate([dstF[:E], epad_d, dstF[E:], epad_d]
                           ).reshape(NC * NS * TPT, TCH)
    pad = jnp.full((NPAD - N,), G, _i32)
    batF = jnp.concatenate([batch1.astype(_i32), pad,
                            batch2.astype(_i32), pad])
    entF = jnp.concatenate([ent1, ent2]).astype(_i32)

    degF, bcntF, entO = _stats_call()(dstR, batF, entF, ent_emb)

    degc = degF.reshape(NC * N, 1)
    xc = jnp.concatenate([x1, x2]).astype(_i32).reshape(NC * N, 1)
    atomP = jnp.pad(atom_emb, ((0, 5), (0, 0)))
    htF, disC = _prep_call(degc, xc, atomP, gcn_W1)

    srcR = jnp.concatenate([srcF[:E], jnp.zeros((EPADG - E,), _i32),
                            srcF[E:], jnp.zeros((EPADG - E,), _i32)]
                           ).reshape(NC * NS * TPT, TCH)
    CF = _cpass_call()(disC.reshape(NC * N), xc.reshape(NC * N), srcR, dstR)
    Cc = jnp.concatenate([CF[:N * NA].reshape(N, NA),
                          CF[CN:CN + N * NA].reshape(N, NA)])
    ht2F = _mid_call(Cc, htF, disC, atomP, gcn_W1,
                     gcn_b1.reshape(1, D), gcn_W2)
    pkF = srcF * 16384 + dstF
    S2 = _msg_call()(ht2F, pkF)
    h2F = _act_call(S2, ht2F, disC, gcn_b2.reshape(1, D))

    zpad = jnp.zeros((NPAD - N, D), _f32)
    h2P = jnp.concatenate([h2F[:N], zpad, h2F[N:], zpad])
    poolF = _pool_call()(h2P, batF)

    return _final_call(
        poolF[:G], poolF[GPAD:GPAD + G],
        bcntF[:G].reshape(G, 1), bcntF[GPAD:GPAD + G].reshape(G, 1),
        entO[:G], entO[G:],
        enc_W1, enc_b1.reshape(1, D), enc_W2, enc_b2.reshape(1, D),
        fc_W, fc_b.reshape(1, D),
        dec_W1[:D], dec_W1[D:], dec_b1.reshape(1, 2 * D),
        dec_W2, dec_b2.reshape(1, 2 * D), dec_W3, dec_b3.reshape(1, D))
